# baseline (speedup 1.0000x reference)
import functools

import jax
import jax.numpy as jnp
from jax import lax
from jax.experimental import pallas as pl
from jax.experimental.pallas import tpu as pltpu
from jax.experimental.pallas import tpu_sc as plsc

_B, _M, _D = 4096, 16384, 256
_TEMP = 0.05
_SCALE = 1.0 / _TEMP
_BT = 1024
_MT = 1024
_LANES = 128


def _lse_body(x_ref, f_ref, lse_ref, acc_ref, xs_ref):
    m = pl.program_id(0)
    b = pl.program_id(1)
    nm = pl.num_programs(0)

    @pl.when((m == 0) & (b == 0))
    def _():
        x = x_ref[...]
        ss = jnp.sum(x * x, axis=1, keepdims=True)
        xs = x * (_SCALE / jnp.maximum(jnp.sqrt(ss), 1e-12))
        xs_ref[...] = xs.astype(jnp.bfloat16)

    rows = pl.ds(b * _BT, _BT)
    xs = xs_ref[rows, :]
    fb = f_ref[...].astype(jnp.bfloat16)
    part = None
    for k in range(_MT // _LANES):
        sk = jax.lax.dot_general(xs, fb[k * _LANES:(k + 1) * _LANES, :],
                                 (((1,), (1,)), ((), ())),
                                 preferred_element_type=jnp.float32)
        ek = jnp.exp(sk.astype(jnp.bfloat16))
        part = ek if part is None else part + ek
    part = part.astype(jnp.float32)

    @pl.when(m == 0)
    def _():
        acc_ref[rows, :] = part

    @pl.when(m > 0)
    def _():
        acc_ref[rows, :] += part

    @pl.when(m == nm - 1)
    def _():
        lse_ref[...] = jnp.log(jnp.sum(acc_ref[rows, :], axis=1))


def _lse_call(inputs, cluster_features):
    nb, nm = _B // _BT, _M // _MT
    return pl.pallas_call(
        _lse_body,
        grid=(nm, nb),
        in_specs=[
            pl.BlockSpec((_B, _D), lambda m, b: (0, 0)),
            pl.BlockSpec((_MT, _D), lambda m, b: (m, 0)),
        ],
        out_specs=pl.BlockSpec((_BT,), lambda m, b: (b,)),
        out_shape=jax.ShapeDtypeStruct((_B,), jnp.float32),
        scratch_shapes=[pltpu.VMEM((_B, _LANES), jnp.float32),
                        pltpu.VMEM((_B, _D), jnp.bfloat16)],
        compiler_params=pltpu.CompilerParams(
            dimension_semantics=("arbitrary", "arbitrary"),
        ),
    )(inputs, cluster_features)


_NC, _NS, _L = 2, 16, 16
_NW = _NC * _NS
_RPW = _B // _NW


def _sc_target_dot(x_hbm, tgt_hbm, feats_hbm, dot_hbm, ss_hbm,
                   idx_v, rows_v, x_v, dot_v, ss_v, sem):
    wid = lax.axis_index("s") * _NC + lax.axis_index("c")
    base = wid * _RPW
    pltpu.sync_copy(tgt_hbm.at[pl.ds(base, _RPW)], idx_v)
    gather = pltpu.async_copy(feats_hbm.at[idx_v], rows_v, sem)
    pltpu.sync_copy(x_hbm.at[pl.ds(base, _RPW), :], x_v)
    gather.wait()

    lanes = lax.iota(jnp.int32, 16)
    zero = jnp.zeros((_L,), jnp.float32)
    for blk in range(_RPW // _L):
        dvec = zero
        svec = zero
        for i in range(_L):
            r = blk * _L + i
            a0 = a1 = s0 = s1 = zero
            for c in range(_D // 32):
                off = c * 32
                f0 = rows_v[r, pl.ds(off, _L)]
                x0 = x_v[r, pl.ds(off, _L)]
                f1 = rows_v[r, pl.ds(off + _L, _L)]
                x1 = x_v[r, pl.ds(off + _L, _L)]
                a0 = a0 + f0 * x0
                a1 = a1 + f1 * x1
                s0 = s0 + x0 * x0
                s1 = s1 + x1 * x1
            dot_r = jnp.sum(a0 + a1)
            ss_r = jnp.sum(s0 + s1)
            dvec = jnp.where(lanes == i, dot_r, dvec)
            svec = jnp.where(lanes == i, ss_r, svec)
        dot_v[pl.ds(blk * _L, _L)] = dvec
        ss_v[pl.ds(blk * _L, _L)] = svec

    pltpu.sync_copy(dot_v, dot_hbm.at[pl.ds(base, _RPW)])
    pltpu.sync_copy(ss_v, ss_hbm.at[pl.ds(base, _RPW)])


def _sc_call(x, tgt, feats):
    sc = pl.kernel(
        _sc_target_dot,
        mesh=plsc.VectorSubcoreMesh(core_axis_name="c", subcore_axis_name="s"),
        out_type=[jax.ShapeDtypeStruct((_B,), jnp.float32),
                  jax.ShapeDtypeStruct((_B,), jnp.float32)],
        scratch_types=[
            pltpu.VMEM((_RPW,), jnp.int32),
            pltpu.VMEM((_RPW, _D), jnp.float32),
            pltpu.VMEM((_RPW, _D), jnp.float32),
            pltpu.VMEM((_RPW,), jnp.float32),
            pltpu.VMEM((_RPW,), jnp.float32),
            pltpu.SemaphoreType.DMA,
        ],
        compiler_params=pltpu.CompilerParams(use_tc_tiling_on_sc=True,
                                             needs_layout_passes=False),
    )
    return sc(x, tgt, feats)


def kernel(inputs, targets, cam_ids, cluster_features):
    rawdot, sumsq = _sc_call(inputs, targets.astype(jnp.int32),
                             cluster_features)
    lse = _lse_call(inputs, cluster_features)
    tgt = rawdot * (_SCALE / jnp.maximum(jnp.sqrt(sumsq), 1e-12))
    return jnp.mean(lse - tgt)


# speedup vs baseline: 1.5555x; 1.5555x over previous
import functools

import jax
import jax.numpy as jnp
from jax import lax
from jax.experimental import pallas as pl
from jax.experimental.pallas import tpu as pltpu
from jax.experimental.pallas import tpu_sc as plsc

_B, _M, _D = 4096, 16384, 256
_TEMP = 0.05
_SCALE = 1.0 / _TEMP
_BT = 2048
_MT = 1024
_LANES = 128


def _lse_body(x_ref, f_ref, lse_ref, acc_ref, xs_ref):
    m = pl.program_id(0)
    b = pl.program_id(1)
    nm = pl.num_programs(0)

    @pl.when((m == 0) & (b == 0))
    def _():
        x = x_ref[...]
        ss = jnp.sum(x * x, axis=1, keepdims=True)
        xs = x * (_SCALE / jnp.maximum(jnp.sqrt(ss), 1e-12))
        xs_ref[...] = xs.astype(jnp.bfloat16)

    rows = pl.ds(b * _BT, _BT)
    s = jax.lax.dot_general(xs_ref[rows, :], f_ref[...].astype(jnp.bfloat16),
                            (((1,), (1,)), ((), ())),
                            preferred_element_type=jnp.float32)
    e = jnp.exp(s.astype(jnp.bfloat16))

    part = e[:, 0:_LANES]
    for k in range(1, _MT // _LANES):
        part = part + e[:, k * _LANES:(k + 1) * _LANES]
    part = part.astype(jnp.float32)

    @pl.when(m == 0)
    def _():
        acc_ref[rows, :] = part

    @pl.when(m > 0)
    def _():
        acc_ref[rows, :] += part

    @pl.when(m == nm - 1)
    def _():
        lse_ref[...] = jnp.log(jnp.sum(acc_ref[rows, :], axis=1))


def _lse_call(inputs, cluster_features):
    nb, nm = _B // _BT, _M // _MT
    return pl.pallas_call(
        _lse_body,
        grid=(nm, nb),
        in_specs=[
            pl.BlockSpec((_B, _D), lambda m, b: (0, 0)),
            pl.BlockSpec((_MT, _D), lambda m, b: (m, 0)),
        ],
        out_specs=pl.BlockSpec((_BT,), lambda m, b: (b,)),
        out_shape=jax.ShapeDtypeStruct((_B,), jnp.float32),
        scratch_shapes=[pltpu.VMEM((_B, _LANES), jnp.float32),
                        pltpu.VMEM((_B, _D), jnp.bfloat16)],
        compiler_params=pltpu.CompilerParams(
            dimension_semantics=("arbitrary", "arbitrary"),
        ),
    )(inputs, cluster_features)


_NC, _NS, _L = 2, 16, 16
_NW = _NC * _NS
_RPW = _B // _NW


def _sc_target_dot(x_hbm, tgt_hbm, feats_hbm, dot_hbm, ss_hbm,
                   idx_v, rows_v, x_v, dot_v, ss_v, sem):
    wid = lax.axis_index("s") * _NC + lax.axis_index("c")
    base = wid * _RPW
    pltpu.sync_copy(tgt_hbm.at[pl.ds(base, _RPW)], idx_v)
    gather = pltpu.async_copy(feats_hbm.at[idx_v], rows_v, sem)
    pltpu.sync_copy(x_hbm.at[pl.ds(base, _RPW), :], x_v)
    gather.wait()

    lanes = lax.iota(jnp.int32, 16)
    zero = jnp.zeros((_L,), jnp.float32)
    for blk in range(_RPW // _L):
        dvec = zero
        svec = zero
        for i in range(_L):
            r = blk * _L + i
            a0 = a1 = s0 = s1 = zero
            for c in range(_D // 32):
                off = c * 32
                f0 = rows_v[r, pl.ds(off, _L)]
                x0 = x_v[r, pl.ds(off, _L)]
                f1 = rows_v[r, pl.ds(off + _L, _L)]
                x1 = x_v[r, pl.ds(off + _L, _L)]
                a0 = a0 + f0 * x0
                a1 = a1 + f1 * x1
                s0 = s0 + x0 * x0
                s1 = s1 + x1 * x1
            dot_r = jnp.sum(a0 + a1)
            ss_r = jnp.sum(s0 + s1)
            dvec = jnp.where(lanes == i, dot_r, dvec)
            svec = jnp.where(lanes == i, ss_r, svec)
        dot_v[pl.ds(blk * _L, _L)] = dvec
        ss_v[pl.ds(blk * _L, _L)] = svec

    pltpu.sync_copy(dot_v, dot_hbm.at[pl.ds(base, _RPW)])
    pltpu.sync_copy(ss_v, ss_hbm.at[pl.ds(base, _RPW)])


def _sc_call(x, tgt, feats):
    sc = pl.kernel(
        _sc_target_dot,
        mesh=plsc.VectorSubcoreMesh(core_axis_name="c", subcore_axis_name="s"),
        out_type=[jax.ShapeDtypeStruct((_B,), jnp.float32),
                  jax.ShapeDtypeStruct((_B,), jnp.float32)],
        scratch_types=[
            pltpu.VMEM((_RPW,), jnp.int32),
            pltpu.VMEM((_RPW, _D), jnp.float32),
            pltpu.VMEM((_RPW, _D), jnp.float32),
            pltpu.VMEM((_RPW,), jnp.float32),
            pltpu.VMEM((_RPW,), jnp.float32),
            pltpu.SemaphoreType.DMA,
        ],
        compiler_params=pltpu.CompilerParams(use_tc_tiling_on_sc=True,
                                             needs_layout_passes=False),
    )
    return sc(x, tgt, feats)


def kernel(inputs, targets, cam_ids, cluster_features):
    rawdot, sumsq = _sc_call(inputs, targets.astype(jnp.int32),
                             cluster_features)
    lse = _lse_call(inputs, cluster_features)
    tgt = rawdot * (_SCALE / jnp.maximum(jnp.sqrt(sumsq), 1e-12))
    return jnp.mean(lse - tgt)


# speedup vs baseline: 1.7028x; 1.0947x over previous
import functools

import jax
import jax.numpy as jnp
from jax import lax
from jax.experimental import pallas as pl
from jax.experimental.pallas import tpu as pltpu
from jax.experimental.pallas import tpu_sc as plsc

_B, _M, _D = 4096, 16384, 256
_TEMP = 0.05
_SCALE = 1.0 / _TEMP
_BT = 4096
_MT = 1024
_LANES = 128


def _lse_body(x_ref, f_ref, lse_ref, acc_ref, xs_ref):
    m = pl.program_id(0)
    b = pl.program_id(1)
    nm = pl.num_programs(0)

    @pl.when((m == 0) & (b == 0))
    def _():
        x = x_ref[...]
        ss = jnp.sum(x * x, axis=1, keepdims=True)
        xs = x * (_SCALE / jnp.maximum(jnp.sqrt(ss), 1e-12))
        xs_ref[...] = xs.astype(jnp.bfloat16)

    rows = pl.ds(b * _BT, _BT)
    s = jax.lax.dot_general(xs_ref[rows, :], f_ref[...].astype(jnp.bfloat16),
                            (((1,), (1,)), ((), ())),
                            preferred_element_type=jnp.float32)
    e = jnp.exp(s.astype(jnp.bfloat16))

    part = e[:, 0:_LANES]
    for k in range(1, _MT // _LANES):
        part = part + e[:, k * _LANES:(k + 1) * _LANES]
    part = part.astype(jnp.float32)

    @pl.when(m == 0)
    def _():
        acc_ref[rows, :] = part

    @pl.when(m > 0)
    def _():
        acc_ref[rows, :] += part

    @pl.when(m == nm - 1)
    def _():
        lse_ref[...] = jnp.log(jnp.sum(acc_ref[rows, :], axis=1))


def _lse_call(inputs, cluster_features):
    nb, nm = _B // _BT, _M // _MT
    return pl.pallas_call(
        _lse_body,
        grid=(nm, nb),
        in_specs=[
            pl.BlockSpec((_B, _D), lambda m, b: (0, 0)),
            pl.BlockSpec((_MT, _D), lambda m, b: (m, 0)),
        ],
        out_specs=pl.BlockSpec((_BT,), lambda m, b: (b,)),
        out_shape=jax.ShapeDtypeStruct((_B,), jnp.float32),
        scratch_shapes=[pltpu.VMEM((_B, _LANES), jnp.float32),
                        pltpu.VMEM((_B, _D), jnp.bfloat16)],
        compiler_params=pltpu.CompilerParams(
            dimension_semantics=("arbitrary", "arbitrary"),
        ),
    )(inputs, cluster_features)


_NC, _NS, _L = 2, 16, 16
_NW = _NC * _NS
_RPW = _B // _NW


def _sc_target_dot(x_hbm, tgt_hbm, feats_hbm, dot_hbm, ss_hbm,
                   idx_v, rows_v, x_v, dot_v, ss_v, sem):
    wid = lax.axis_index("s") * _NC + lax.axis_index("c")
    base = wid * _RPW
    pltpu.sync_copy(tgt_hbm.at[pl.ds(base, _RPW)], idx_v)
    gather = pltpu.async_copy(feats_hbm.at[idx_v], rows_v, sem)
    pltpu.sync_copy(x_hbm.at[pl.ds(base, _RPW), :], x_v)
    gather.wait()

    lanes = lax.iota(jnp.int32, 16)
    zero = jnp.zeros((_L,), jnp.float32)
    for blk in range(_RPW // _L):
        dvec = zero
        svec = zero
        for i in range(_L):
            r = blk * _L + i
            a0 = a1 = s0 = s1 = zero
            for c in range(_D // 32):
                off = c * 32
                f0 = rows_v[r, pl.ds(off, _L)]
                x0 = x_v[r, pl.ds(off, _L)]
                f1 = rows_v[r, pl.ds(off + _L, _L)]
                x1 = x_v[r, pl.ds(off + _L, _L)]
                a0 = a0 + f0 * x0
                a1 = a1 + f1 * x1
                s0 = s0 + x0 * x0
                s1 = s1 + x1 * x1
            dot_r = jnp.sum(a0 + a1)
            ss_r = jnp.sum(s0 + s1)
            dvec = jnp.where(lanes == i, dot_r, dvec)
            svec = jnp.where(lanes == i, ss_r, svec)
        dot_v[pl.ds(blk * _L, _L)] = dvec
        ss_v[pl.ds(blk * _L, _L)] = svec

    pltpu.sync_copy(dot_v, dot_hbm.at[pl.ds(base, _RPW)])
    pltpu.sync_copy(ss_v, ss_hbm.at[pl.ds(base, _RPW)])


def _sc_call(x, tgt, feats):
    sc = pl.kernel(
        _sc_target_dot,
        mesh=plsc.VectorSubcoreMesh(core_axis_name="c", subcore_axis_name="s"),
        out_type=[jax.ShapeDtypeStruct((_B,), jnp.float32),
                  jax.ShapeDtypeStruct((_B,), jnp.float32)],
        scratch_types=[
            pltpu.VMEM((_RPW,), jnp.int32),
            pltpu.VMEM((_RPW, _D), jnp.float32),
            pltpu.VMEM((_RPW, _D), jnp.float32),
            pltpu.VMEM((_RPW,), jnp.float32),
            pltpu.VMEM((_RPW,), jnp.float32),
            pltpu.SemaphoreType.DMA,
        ],
        compiler_params=pltpu.CompilerParams(use_tc_tiling_on_sc=True,
                                             needs_layout_passes=False),
    )
    return sc(x, tgt, feats)


def kernel(inputs, targets, cam_ids, cluster_features):
    rawdot, sumsq = _sc_call(inputs, targets.astype(jnp.int32),
                             cluster_features)
    lse = _lse_call(inputs, cluster_features)
    tgt = rawdot * (_SCALE / jnp.maximum(jnp.sqrt(sumsq), 1e-12))
    return jnp.mean(lse - tgt)


# speedup vs baseline: 1.7847x; 1.0481x over previous
import functools

import jax
import jax.numpy as jnp
from jax import lax
from jax.experimental import pallas as pl
from jax.experimental.pallas import tpu as pltpu
from jax.experimental.pallas import tpu_sc as plsc

_B, _M, _D = 4096, 16384, 256
_TEMP = 0.05
_SCALE = 1.0 / _TEMP
_BT = 4096
_MT = 2048
_LANES = 128


def _lse_body(x_ref, f_ref, lse_ref, acc_ref, xs_ref):
    m = pl.program_id(0)
    b = pl.program_id(1)
    nm = pl.num_programs(0)

    @pl.when((m == 0) & (b == 0))
    def _():
        x = x_ref[...]
        ss = jnp.sum(x * x, axis=1, keepdims=True)
        xs = x * (_SCALE / jnp.maximum(jnp.sqrt(ss), 1e-12))
        xs_ref[...] = xs.astype(jnp.bfloat16)

    rows = pl.ds(b * _BT, _BT)
    s = jax.lax.dot_general(xs_ref[rows, :], f_ref[...].astype(jnp.bfloat16),
                            (((1,), (1,)), ((), ())),
                            preferred_element_type=jnp.float32)
    e = jnp.exp(s.astype(jnp.bfloat16))

    part = e[:, 0:_LANES]
    for k in range(1, _MT // _LANES):
        part = part + e[:, k * _LANES:(k + 1) * _LANES]
    part = part.astype(jnp.float32)

    @pl.when(m == 0)
    def _():
        acc_ref[rows, :] = part

    @pl.when(m > 0)
    def _():
        acc_ref[rows, :] += part

    @pl.when(m == nm - 1)
    def _():
        lse_ref[...] = jnp.log(jnp.sum(acc_ref[rows, :], axis=1))


def _lse_call(inputs, cluster_features):
    nb, nm = _B // _BT, _M // _MT
    return pl.pallas_call(
        _lse_body,
        grid=(nm, nb),
        in_specs=[
            pl.BlockSpec((_B, _D), lambda m, b: (0, 0)),
            pl.BlockSpec((_MT, _D), lambda m, b: (m, 0)),
        ],
        out_specs=pl.BlockSpec((_BT,), lambda m, b: (b,)),
        out_shape=jax.ShapeDtypeStruct((_B,), jnp.float32),
        scratch_shapes=[pltpu.VMEM((_B, _LANES), jnp.float32),
                        pltpu.VMEM((_B, _D), jnp.bfloat16)],
        compiler_params=pltpu.CompilerParams(
            dimension_semantics=("arbitrary", "arbitrary"),
        ),
    )(inputs, cluster_features)


_NC, _NS, _L = 2, 16, 16
_NW = _NC * _NS
_RPW = _B // _NW


def _sc_target_dot(x_hbm, tgt_hbm, feats_hbm, dot_hbm, ss_hbm,
                   idx_v, rows_v, x_v, dot_v, ss_v, sem):
    wid = lax.axis_index("s") * _NC + lax.axis_index("c")
    base = wid * _RPW
    pltpu.sync_copy(tgt_hbm.at[pl.ds(base, _RPW)], idx_v)
    gather = pltpu.async_copy(feats_hbm.at[idx_v], rows_v, sem)
    pltpu.sync_copy(x_hbm.at[pl.ds(base, _RPW), :], x_v)
    gather.wait()

    lanes = lax.iota(jnp.int32, 16)
    zero = jnp.zeros((_L,), jnp.float32)
    for blk in range(_RPW // _L):
        dvec = zero
        svec = zero
        for i in range(_L):
            r = blk * _L + i
            a0 = a1 = s0 = s1 = zero
            for c in range(_D // 32):
                off = c * 32
                f0 = rows_v[r, pl.ds(off, _L)]
                x0 = x_v[r, pl.ds(off, _L)]
                f1 = rows_v[r, pl.ds(off + _L, _L)]
                x1 = x_v[r, pl.ds(off + _L, _L)]
                a0 = a0 + f0 * x0
                a1 = a1 + f1 * x1
                s0 = s0 + x0 * x0
                s1 = s1 + x1 * x1
            dot_r = jnp.sum(a0 + a1)
            ss_r = jnp.sum(s0 + s1)
            dvec = jnp.where(lanes == i, dot_r, dvec)
            svec = jnp.where(lanes == i, ss_r, svec)
        dot_v[pl.ds(blk * _L, _L)] = dvec
        ss_v[pl.ds(blk * _L, _L)] = svec

    pltpu.sync_copy(dot_v, dot_hbm.at[pl.ds(base, _RPW)])
    pltpu.sync_copy(ss_v, ss_hbm.at[pl.ds(base, _RPW)])


def _sc_call(x, tgt, feats):
    sc = pl.kernel(
        _sc_target_dot,
        mesh=plsc.VectorSubcoreMesh(core_axis_name="c", subcore_axis_name="s"),
        out_type=[jax.ShapeDtypeStruct((_B,), jnp.float32),
                  jax.ShapeDtypeStruct((_B,), jnp.float32)],
        scratch_types=[
            pltpu.VMEM((_RPW,), jnp.int32),
            pltpu.VMEM((_RPW, _D), jnp.float32),
            pltpu.VMEM((_RPW, _D), jnp.float32),
            pltpu.VMEM((_RPW,), jnp.float32),
            pltpu.VMEM((_RPW,), jnp.float32),
            pltpu.SemaphoreType.DMA,
        ],
        compiler_params=pltpu.CompilerParams(use_tc_tiling_on_sc=True,
                                             needs_layout_passes=False),
    )
    return sc(x, tgt, feats)


def kernel(inputs, targets, cam_ids, cluster_features):
    rawdot, sumsq = _sc_call(inputs, targets.astype(jnp.int32),
                             cluster_features)
    lse = _lse_call(inputs, cluster_features)
    tgt = rawdot * (_SCALE / jnp.maximum(jnp.sqrt(sumsq), 1e-12))
    return jnp.mean(lse - tgt)
